# Initial kernel scaffold; baseline (speedup 1.0000x reference)
import functools

import jax
import jax.numpy as jnp
from jax import lax
from jax.experimental import pallas as pl
from jax.experimental.pallas import tpu as pltpu
from jax.experimental.pallas import tpu_sc as plsc

N = 10000
NP = 10240
D = 256
DH = 128
KITER = 10
E = 160000
NTILES = 16
RPT = NP // NTILES
NCH = 80
EPT = NCH * 128
EPAD = NTILES * EPT
PADN = N
NBLK = RPT // 128


def _rsqrt(x):
    i = plsc.bitcast(x, jnp.int32)
    i = jnp.int32(0x5F3759DF) - lax.shift_right_arithmetic(i, 1)
    y = plsc.bitcast(i, jnp.float32)
    for _ in range(3):
        y = y * (1.5 - 0.5 * x * y * y)
    return y


def _splat(ref, n):
    return plsc.load_gather(ref, [jnp.full((16,), n, jnp.int32)])


@functools.partial(
    pl.kernel,
    out_type=(
        jax.ShapeDtypeStruct((2 * NP, DH), jnp.float32),
        jax.ShapeDtypeStruct((2 * NP, DH), jnp.float32),
        jax.ShapeDtypeStruct((2 * NP, DH), jnp.float32),
    ),
    mesh=plsc.VectorSubcoreMesh(core_axis_name="c", subcore_axis_name="s"),
    scratch_types=[
        pltpu.VMEM_SHARED((NP, DH), jnp.float32),
        pltpu.VMEM_SHARED((NP, 16), jnp.float32),
        pltpu.VMEM_SHARED((NP, 16), jnp.float32),
        pltpu.VMEM((NCH, 128), jnp.int32),
        pltpu.VMEM((NCH, 128), jnp.int32),
        pltpu.VMEM((128, DH), jnp.float32),
        pltpu.VMEM((128, DH), jnp.float32),
        pltpu.VMEM((128, 16), jnp.float32),
        pltpu.VMEM((RPT, 16), jnp.float32),
        pltpu.VMEM((RPT, 16), jnp.float32),
        pltpu.VMEM((RPT,), jnp.float32),
        pltpu.VMEM((RPT,), jnp.float32),
        pltpu.VMEM((RPT,), jnp.float32),
        pltpu.SemaphoreType.DMA,
        pltpu.SemaphoreType.DMA,
    ],
)
def _appnp_sc(feat_hbm, src_hbm, dst_hbm, out_hbm, g_hbm, a2_hbm,
              accum, hist_s, hist_d, src_idx, dst_idx, bufA, bufB,
              ones16, zbuf, hbuf, nsrc, ndst, ndinv, semA, semB):
    c = lax.axis_index("c")
    s = lax.axis_index("s")
    rb_sp = s * RPT
    rb_hbm = c * NP + rb_sp
    iota16 = lax.iota(jnp.int32, 16)
    zeros16i = jnp.zeros((16,), jnp.int32)

    pltpu.sync_copy(src_hbm.at[s], src_idx)
    pltpu.sync_copy(dst_hbm.at[s], dst_idx)

    @pl.loop(0, 128)
    def _(i):
        ones16[i] = jnp.ones((16,), jnp.float32)

    @pl.loop(0, RPT)
    def _(i):
        zbuf[i] = jnp.zeros((16,), jnp.float32)

    pltpu.sync_copy(zbuf, hist_s.at[pl.ds(rb_sp, RPT)])
    pltpu.sync_copy(zbuf, hist_d.at[pl.ds(rb_sp, RPT)])
    plsc.subcore_barrier()

    @pl.loop(0, NCH)
    def _(j):
        pltpu.sync_copy(ones16, hist_s.at[src_idx.at[j]], add=True)
        pltpu.sync_copy(ones16, hist_d.at[dst_idx.at[j]], add=True)
    plsc.subcore_barrier()

    pltpu.sync_copy(hist_s.at[pl.ds(rb_sp, RPT)], hbuf)

    @pl.loop(0, RPT // 16)
    def _(i):
        cnt = plsc.load_gather(hbuf, [i * 16 + iota16, zeros16i])
        nsrc[pl.ds(i * 16, 16)] = _rsqrt(jnp.maximum(cnt, 1.0))

    pltpu.sync_copy(hist_d.at[pl.ds(rb_sp, RPT)], hbuf)

    @pl.loop(0, RPT // 16)
    def _(i):
        cnt = jnp.maximum(plsc.load_gather(hbuf, [i * 16 + iota16, zeros16i]), 1.0)
        r = _rsqrt(cnt)
        ndst[pl.ds(i * 16, 16)] = r
        ndinv[pl.ds(i * 16, 16)] = cnt * r

    for blk in range(NBLK):
        pltpu.sync_copy(feat_hbm.at[pl.ds(rb_hbm + blk * 128, 128)], bufA)

        @pl.loop(0, 128)
        def _(i, blk=blk):
            nl = blk * 128 + i
            ws = _splat(nsrc, nl)
            wa = _splat(ndinv, nl) * (1.0 / 9.0)
            for q in range(8):
                v = bufA[i, pl.ds(q * 16, 16)]
                bufB[i, pl.ds(q * 16, 16)] = v * ws
                bufA[i, pl.ds(q * 16, 16)] = v * wa

        pltpu.sync_copy(bufB, g_hbm.at[pl.ds(rb_hbm + blk * 128, 128)])
        pltpu.sync_copy(bufA, a2_hbm.at[pl.ds(rb_hbm + blk * 128, 128)])

    off = c * NP

    @pl.loop(0, NCH)
    def _(j):
        for q in range(8):
            v = src_idx[j, pl.ds(q * 16, 16)]
            src_idx[j, pl.ds(q * 16, 16)] = v + off

    def seed_accum():
        pltpu.sync_copy(a2_hbm.at[pl.ds(rb_hbm, RPT)],
                        accum.at[pl.ds(rb_sp, RPT)])
        plsc.subcore_barrier()

    def edge_sweep():
        pltpu.async_copy(g_hbm.at[src_idx.at[0]], bufA, semA)

        @pl.loop(0, NCH, step=2)
        def _(j):
            pltpu.async_copy(g_hbm.at[src_idx.at[j + 1]], bufB, semB)
            pltpu.make_async_copy(g_hbm.at[src_idx.at[j]], bufA, semA).wait()
            pltpu.sync_copy(bufA, accum.at[dst_idx.at[j]], add=True)

            @pl.when(j + 2 < NCH)
            def _():
                pltpu.async_copy(g_hbm.at[src_idx.at[j + 2]], bufA, semA)

            pltpu.make_async_copy(g_hbm.at[src_idx.at[j + 1]], bufB, semB).wait()
            pltpu.sync_copy(bufB, accum.at[dst_idx.at[j + 1]], add=True)

        plsc.subcore_barrier()

    def finish_round(dst_arr, rescale_for_next):
        for blk in range(NBLK):
            pltpu.sync_copy(accum.at[pl.ds(rb_sp + blk * 128, 128)], bufA)

            @pl.loop(0, 128)
            def _(i, blk=blk):
                nl = blk * 128 + i
                w = _splat(ndst, nl) * 0.9
                if rescale_for_next:
                    w = w * _splat(nsrc, nl)
                for q in range(8):
                    bufB[i, pl.ds(q * 16, 16)] = bufA[i, pl.ds(q * 16, 16)] * w

            pltpu.sync_copy(bufB, dst_arr.at[pl.ds(rb_hbm + blk * 128, 128)])

    @pl.loop(0, KITER - 1)
    def _(k):
        seed_accum()
        edge_sweep()
        finish_round(g_hbm, True)

    seed_accum()
    edge_sweep()
    finish_round(out_hbm, False)


def kernel(feat, edge_index):
    feat = feat.astype(jnp.float32)
    f2 = feat.reshape(N, 2, DH).transpose(1, 0, 2)
    f2 = jnp.pad(f2, ((0, 0), (0, NP - N), (0, 0)))
    feat_flat = f2.reshape(2 * NP, DH)
    src = edge_index[0].astype(jnp.int32)
    dst = edge_index[1].astype(jnp.int32)
    src_p = jnp.pad(src, (0, EPAD - E), constant_values=PADN)
    dst_p = jnp.pad(dst, (0, EPAD - E), constant_values=PADN)
    src_p = src_p.reshape(NTILES, NCH, 128)
    dst_p = dst_p.reshape(NTILES, NCH, 128)
    out, _, _ = _appnp_sc(feat_flat, src_p, dst_p)
    return jnp.concatenate([out[0:N], out[NP:NP + N]], axis=1)


# speedup vs baseline: 3.0104x; 3.0104x over previous
import functools

import jax
import jax.numpy as jnp
from jax import lax
from jax.experimental import pallas as pl
from jax.experimental.pallas import tpu as pltpu
from jax.experimental.pallas import tpu_sc as plsc

N = 10000
NP = 10240
D = 256
DH = 128
KITER = 10
E = 160000
NTILES = 16
RPT = NP // NTILES
CH = 64
NCH = 160
EPT = NCH * CH
EPAD = NTILES * EPT
PADN = N
NBLK = RPT // CH


def _rsqrt(x):
    i = plsc.bitcast(x, jnp.int32)
    i = jnp.int32(0x5F3759DF) - lax.shift_right_arithmetic(i, 1)
    y = plsc.bitcast(i, jnp.float32)
    for _ in range(3):
        y = y * (1.5 - 0.5 * x * y * y)
    return y


def _splat(ref, n):
    return plsc.load_gather(ref, [jnp.full((16,), n, jnp.int32)])


@functools.partial(
    pl.kernel,
    out_type=(
        jax.ShapeDtypeStruct((2 * NP, DH), jnp.float32),
        jax.ShapeDtypeStruct((2 * NP, DH), jnp.float32),
        jax.ShapeDtypeStruct((2 * NP, DH), jnp.float32),
    ),
    mesh=plsc.VectorSubcoreMesh(core_axis_name="c", subcore_axis_name="s"),
    compiler_params=pltpu.CompilerParams(
        needs_layout_passes=False, use_tc_tiling_on_sc=False),
    scratch_types=[
        pltpu.VMEM_SHARED((NP, DH), jnp.float32),
        pltpu.VMEM_SHARED((NP,), jnp.float32),
        pltpu.VMEM_SHARED((NP,), jnp.float32),
        pltpu.VMEM((NCH, CH), jnp.int32),
        pltpu.VMEM((NCH, CH), jnp.int32),
        pltpu.VMEM((CH, DH), jnp.float32),
        pltpu.VMEM((CH, DH), jnp.float32),
        pltpu.VMEM((CH,), jnp.float32),
        pltpu.VMEM((RPT,), jnp.float32),
        pltpu.VMEM((RPT,), jnp.float32),
        pltpu.VMEM((RPT,), jnp.float32),
        pltpu.VMEM((RPT,), jnp.float32),
        pltpu.SemaphoreType.DMA,
        pltpu.SemaphoreType.DMA,
    ],
)
def _appnp_sc(feat_hbm, src_hbm, dst_hbm, out_hbm, g_hbm, a2_hbm,
              accum, hist_s, hist_d, src_idx, dst_idx, bufA, bufB,
              ones16, hbuf, nsrc, ndst, ndinv, semA, semB):
    c = lax.axis_index("c")
    s = lax.axis_index("s")
    rb_sp = s * RPT
    rb_hbm = c * NP + rb_sp
    pltpu.sync_copy(src_hbm.at[s], src_idx)
    pltpu.sync_copy(dst_hbm.at[s], dst_idx)

    @pl.loop(0, CH // 16)
    def _(i):
        ones16[pl.ds(i * 16, 16)] = jnp.ones((16,), jnp.float32)

    @pl.loop(0, RPT // 16)
    def _(i):
        hbuf[pl.ds(i * 16, 16)] = jnp.zeros((16,), jnp.float32)

    pltpu.sync_copy(hbuf, hist_s.at[pl.ds(rb_sp, RPT)])
    pltpu.sync_copy(hbuf, hist_d.at[pl.ds(rb_sp, RPT)])
    plsc.subcore_barrier()

    @pl.loop(0, NCH)
    def _(j):
        pltpu.sync_copy(ones16, hist_s.at[src_idx.at[j]], add=True)
        pltpu.sync_copy(ones16, hist_d.at[dst_idx.at[j]], add=True)
    plsc.subcore_barrier()

    pltpu.sync_copy(hist_s.at[pl.ds(rb_sp, RPT)], hbuf)

    @pl.loop(0, RPT // 16)
    def _(i):
        cnt = hbuf[pl.ds(i * 16, 16)]
        nsrc[pl.ds(i * 16, 16)] = _rsqrt(jnp.maximum(cnt, 1.0))

    pltpu.sync_copy(hist_d.at[pl.ds(rb_sp, RPT)], hbuf)

    @pl.loop(0, RPT // 16)
    def _(i):
        cnt = jnp.maximum(hbuf[pl.ds(i * 16, 16)], 1.0)
        r = _rsqrt(cnt)
        ndst[pl.ds(i * 16, 16)] = r
        ndinv[pl.ds(i * 16, 16)] = cnt * r

    for blk in range(NBLK):
        pltpu.sync_copy(feat_hbm.at[pl.ds(rb_hbm + blk * CH, CH)], bufA)

        @pl.loop(0, CH)
        def _(i, blk=blk):
            nl = blk * CH + i
            ws = _splat(nsrc, nl)
            wa = _splat(ndinv, nl) * (1.0 / 9.0)
            for q in range(8):
                v = bufA[i, pl.ds(q * 16, 16)]
                bufB[i, pl.ds(q * 16, 16)] = v * ws
                bufA[i, pl.ds(q * 16, 16)] = v * wa

        pltpu.sync_copy(bufB, g_hbm.at[pl.ds(rb_hbm + blk * CH, CH)])
        pltpu.sync_copy(bufA, a2_hbm.at[pl.ds(rb_hbm + blk * CH, CH)])

    off = c * NP

    @pl.loop(0, NCH)
    def _(j):
        for q in range(CH // 16):
            v = src_idx[j, pl.ds(q * 16, 16)]
            src_idx[j, pl.ds(q * 16, 16)] = v + off

    def seed_accum():
        pltpu.sync_copy(a2_hbm.at[pl.ds(rb_hbm, RPT)],
                        accum.at[pl.ds(rb_sp, RPT)])
        plsc.subcore_barrier()

    def edge_sweep():
        pltpu.async_copy(g_hbm.at[src_idx.at[0]], bufA, semA)

        @pl.loop(0, NCH, step=2)
        def _(j):
            pltpu.async_copy(g_hbm.at[src_idx.at[j + 1]], bufB, semB)
            pltpu.make_async_copy(g_hbm.at[src_idx.at[j]], bufA, semA).wait()
            pltpu.sync_copy(bufA, accum.at[dst_idx.at[j]], add=True)

            @pl.when(j + 2 < NCH)
            def _():
                pltpu.async_copy(g_hbm.at[src_idx.at[j + 2]], bufA, semA)

            pltpu.make_async_copy(g_hbm.at[src_idx.at[j + 1]], bufB, semB).wait()
            pltpu.sync_copy(bufB, accum.at[dst_idx.at[j + 1]], add=True)

        plsc.subcore_barrier()

    def finish_round(dst_arr, rescale_for_next):
        for blk in range(NBLK):
            pltpu.sync_copy(accum.at[pl.ds(rb_sp + blk * CH, CH)], bufA)

            @pl.loop(0, CH)
            def _(i, blk=blk):
                nl = blk * CH + i
                w = _splat(ndst, nl) * 0.9
                if rescale_for_next:
                    w = w * _splat(nsrc, nl)
                for q in range(8):
                    bufB[i, pl.ds(q * 16, 16)] = bufA[i, pl.ds(q * 16, 16)] * w

            pltpu.sync_copy(bufB, dst_arr.at[pl.ds(rb_hbm + blk * CH, CH)])

    @pl.loop(0, KITER - 1)
    def _(k):
        seed_accum()
        edge_sweep()
        finish_round(g_hbm, True)

    seed_accum()
    edge_sweep()
    finish_round(out_hbm, False)


def kernel(feat, edge_index):
    feat = feat.astype(jnp.float32)
    f2 = feat.reshape(N, 2, DH).transpose(1, 0, 2)
    f2 = jnp.pad(f2, ((0, 0), (0, NP - N), (0, 0)))
    feat_flat = f2.reshape(2 * NP, DH)
    src = edge_index[0].astype(jnp.int32)
    dst = edge_index[1].astype(jnp.int32)
    src_p = jnp.pad(src, (0, EPAD - E), constant_values=PADN)
    dst_p = jnp.pad(dst, (0, EPAD - E), constant_values=PADN)
    src_p = src_p.reshape(NTILES, NCH, CH)
    dst_p = dst_p.reshape(NTILES, NCH, CH)
    out, _, _ = _appnp_sc(feat_flat, src_p, dst_p)
    return jnp.concatenate([out[0:N], out[NP:NP + N]], axis=1)
